# baseline (speedup 1.0000x reference)
import functools

import jax
import jax.numpy as jnp
from jax import lax
from jax.experimental import pallas as pl
from jax.experimental.pallas import tpu as pltpu
from jax.experimental.pallas import tpu_sc as plsc

_N = 10000
_E = 160000
_D = 256
_CH = 80
_NCH = _E // _CH
_F32 = jnp.float32

_mesh = plsc.VectorSubcoreMesh(core_axis_name="c", subcore_axis_name="s")
_scp = pltpu.CompilerParams(needs_layout_passes=False)


_GDN = lax.GatherDimensionNumbers(offset_dims=(), collapsed_slice_dims=(0,),
                                 start_index_map=(0,))


def _bcast(vec, j):
    idx = jnp.full((16, 1), j, jnp.int32)
    return lax.gather(vec, idx, _GDN, (1,),
                      mode=lax.GatherScatterMode.PROMISE_IN_BOUNDS)


def _mm_body(h_ref, wl_ref, bl_ref, wr_ref, br_ref,
             xl_ref, xr_ref, xlo_ref, xhi_ref):
    h = h_ref[...]
    yl = jnp.dot(h, wl_ref[...], preferred_element_type=_F32) + bl_ref[...]
    yr = jnp.dot(h, wr_ref[...], preferred_element_type=_F32) + br_ref[...]
    xl_ref[...] = yl.astype(jnp.bfloat16)
    xr_ref[...] = yr.astype(jnp.bfloat16)
    xlo_ref[...] = yl[:, :128]
    xhi_ref[...] = yl[:, 128:]


def _mm(h, Wl, bl, Wr, br):
    BN = 1000
    return pl.pallas_call(
        _mm_body,
        grid=(_N // BN,),
        in_specs=[pl.BlockSpec((BN, _D), lambda i: (i, 0)),
                  pl.BlockSpec((_D, _D), lambda i: (0, 0)),
                  pl.BlockSpec((1, _D), lambda i: (0, 0)),
                  pl.BlockSpec((_D, _D), lambda i: (0, 0)),
                  pl.BlockSpec((1, _D), lambda i: (0, 0))],
        out_specs=[pl.BlockSpec((BN, _D), lambda i: (i, 0)),
                   pl.BlockSpec((BN, _D), lambda i: (i, 0)),
                   pl.BlockSpec((BN, 128), lambda i: (i, 0)),
                   pl.BlockSpec((BN, 128), lambda i: (i, 0))],
        out_shape=[jax.ShapeDtypeStruct((_N, _D), jnp.bfloat16),
                   jax.ShapeDtypeStruct((_N, _D), jnp.bfloat16),
                   jax.ShapeDtypeStruct((_N, 128), _F32),
                   jax.ShapeDtypeStruct((_N, 128), _F32)],
    )(h, Wl, bl.reshape(1, _D), Wr, br.reshape(1, _D))


@functools.partial(
    pl.kernel, mesh=_mesh, compiler_params=_scp,
    out_type=[jax.ShapeDtypeStruct((_NCH, _CH), _F32),
              jax.ShapeDtypeStruct((2, _N), _F32)],
    scratch_types=[
        pltpu.VMEM((64, _CH), jnp.int32),
        pltpu.VMEM((64, _CH), jnp.int32),
        pltpu.VMEM((64, _CH), _F32),
        pltpu.VMEM((64, _CH), _F32),
        pltpu.VMEM((64, _CH), _F32),
        pltpu.VMEM((_CH, 128), jnp.int32),
        pltpu.VMEM((_CH, 128), jnp.int32),
        pltpu.VMEM((_CH, 128), jnp.int32),
        pltpu.VMEM((_CH, 128), jnp.int32),
        pltpu.VMEM((4, _D), _F32),
        pltpu.VMEM_SHARED((_N,), _F32),
        pltpu.SemaphoreType.DMA,
        pltpu.SemaphoreType.DMA,
        pltpu.SemaphoreType.DMA,
        pltpu.SemaphoreType.DMA,
        pltpu.SemaphoreType.DMA,
        pltpu.SemaphoreType.DMA,
        pltpu.SemaphoreType.DMA,
    ])
def _pass1(xl_hbm, xr_hbm, src_hbm, dst_hbm, ea0_hbm, ea1_hbm, wa_hbm,
           zf_hbm, ex_hbm, den_hbm,
           src_blk, dst_blk, ea0_blk, ea1_blk, ex_blk,
           xl0, xl1, xr0, xr1, wa_v,
           shared_den, sem_i, sem_xl0, sem_xl1, sem_xr0, sem_xr1,
           sem_d0, sem_d1):
    cid = lax.axis_index("c")
    sid = lax.axis_index("s")
    wid = sid * 2 + cid
    nt = jnp.where(wid < 26, 64, 56)
    cstart = wid * 64 - 8 * jnp.maximum(wid - 26, 0)

    @pl.when(sid == 0)
    def _zero():
        pltpu.sync_copy(zf_hbm, shared_den)

    pairs = ((src_hbm, src_blk), (dst_hbm, dst_blk),
             (ea0_hbm, ea0_blk), (ea1_hbm, ea1_blk))

    @pl.when(wid < 26)
    def _copy_full():
        waits = [pltpu.async_copy(h.at[pl.ds(cstart, 64)],
                                  v.at[pl.ds(0, 64)], sem_i)
                 for h, v in pairs]
        for w in waits:
            w.wait()

    @pl.when(wid >= 26)
    def _copy_small():
        waits = [pltpu.async_copy(h.at[pl.ds(cstart, 56)],
                                  v.at[pl.ds(0, 56)], sem_i)
                 for h, v in pairs]
        for w in waits:
            w.wait()

    pltpu.sync_copy(wa_hbm, wa_v)

    plsc.subcore_barrier()

    we0 = [wa_v[0, pl.ds(v * 16, 16)] for v in range(16)]
    we1 = [wa_v[1, pl.ds(v * 16, 16)] for v in range(16)]
    att6 = [wa_v[2, pl.ds(v * 16, 16)] for v in range(16)]
    att4 = [wa_v[3, pl.ds(v * 16, 16)] for v in range(16)]
    lane = lax.iota(jnp.int32, 16)

    bufs = ((xl0, xr0, sem_xl0, sem_xr0), (xl1, xr1, sem_xl1, sem_xr1))
    sem_d = (sem_d0, sem_d1)

    def issue(t, b):
        xlb, xrb, sl, sr = bufs[b]
        pltpu.async_copy(xl_hbm.at[src_blk.at[t]], xlb, sl)
        pltpu.async_copy(xr_hbm.at[dst_blk.at[t]], xrb, sr)

    def drain(t, b):
        xlb, xrb, sl, sr = bufs[b]
        pltpu.make_async_copy(xl_hbm.at[src_blk.at[t]], xlb, sl).wait()
        pltpu.make_async_copy(xr_hbm.at[dst_blk.at[t]], xrb, sr).wait()

    def compute(t, b):
        xlb, xrb, _, _ = bufs[b]

        def group(g, _):
            gb = g * 16
            ea0g = ea0_blk[t, pl.ds(gb, 16)]
            ea1g = ea1_blk[t, pl.ds(gb, 16)]
            alpha_g = jnp.zeros((16,), _F32)
            for j in range(16):
                e = gb + j
                ea0 = _bcast(ea0g, j)
                ea1 = _bcast(ea1g, j)
                acc = jnp.zeros((16,), _F32)
                for v in range(8):
                    xlv = plsc.bitcast(xlb[e, pl.ds(v * 16, 16)], jnp.bfloat16)
                    xrv = plsc.bitcast(xrb[e, pl.ds(v * 16, 16)], jnp.bfloat16)
                    xle, xlo_ = plsc.unpack(
                        xlv, format=plsc.PackFormat.INTERLEAVED,
                        preferred_element_type=_F32)
                    xre, xro = plsc.unpack(
                        xrv, format=plsc.PackFormat.INTERLEAVED,
                        preferred_element_type=_F32)
                    for h, (xlh, xrh) in enumerate(((xle, xre), (xlo_, xro))):
                        k2 = 2 * v + h
                        m = xlh + xrh
                        m = ea0 * we0[k2] + m
                        m = ea1 * we1[k2] + m
                        acc = m * att6[k2] + acc
                        acc = jnp.abs(m) * att4[k2] + acc
                alpha = plsc.cumsum(acc)[15]
                alpha_g = jnp.where(lane == j, alpha, alpha_g)
            ex_blk[t, pl.ds(gb, 16)] = jnp.exp(alpha_g)
            return 0

        lax.fori_loop(0, _CH // 16, group, 0)
        pltpu.async_copy(ex_blk.at[t], shared_den.at[dst_blk.at[t]],
                         sem_d[b], add=True)

    def drain_denom(t, b):
        pltpu.make_async_copy(ex_blk.at[t], shared_den.at[dst_blk.at[t]],
                              sem_d[b]).wait()

    issue(0, 0)

    def outer(t2, _):
        for b in range(2):
            t = 2 * t2 + b

            @pl.when(t < nt)
            def _():
                @pl.when(t + 1 < nt)
                def _():
                    issue(t + 1, 1 - b)

                drain(t, b)

                @pl.when(t >= 2)
                def _():
                    drain_denom(t - 2, b)

                compute(t, b)
        return 0

    lax.fori_loop(0, 32, outer, 0)
    drain_denom(nt - 2, 0)
    drain_denom(nt - 1, 1)

    @pl.when(wid < 26)
    def _ex_full():
        pltpu.sync_copy(ex_blk.at[pl.ds(0, 64)],
                        ex_hbm.at[pl.ds(cstart, 64)])

    @pl.when(wid >= 26)
    def _ex_small():
        pltpu.sync_copy(ex_blk.at[pl.ds(0, 56)],
                        ex_hbm.at[pl.ds(cstart, 56)])

    plsc.subcore_barrier()

    @pl.when(sid == 0)
    def _out():
        pltpu.sync_copy(shared_den, den_hbm.at[cid])


@functools.partial(
    pl.kernel, mesh=_mesh, compiler_params=_scp,
    out_type=jax.ShapeDtypeStruct((2, _N, 128), _F32),
    scratch_types=[
        pltpu.VMEM((32, _CH), jnp.int32),
        pltpu.VMEM((32, _CH), jnp.int32),
        pltpu.VMEM((_CH,), _F32),
        pltpu.VMEM((_CH,), _F32),
        pltpu.VMEM((_CH, 128), _F32),
        pltpu.VMEM((_CH, 128), _F32),
        pltpu.VMEM_SHARED((_N, 128), _F32),
        pltpu.SemaphoreType.DMA,
        pltpu.SemaphoreType.DMA,
        pltpu.SemaphoreType.DMA,
        pltpu.SemaphoreType.DMA,
        pltpu.SemaphoreType.DMA,
        pltpu.SemaphoreType.DMA,
        pltpu.SemaphoreType.DMA,
    ])
def _pass2(xlo_hbm, xhi_hbm, src_hbm, dst_hbm, ex_hbm, z2_hbm,
           out_hbm,
           src_blk, dst_blk, exb0, exb1, rows0, rows1, shared_acc,
           sem_i, sem_g0, sem_g1, sem_e0, sem_e1, sem_s0, sem_s1):
    cid = lax.axis_index("c")
    sid = lax.axis_index("s")
    cs = sid * 128 - 8 * jnp.maximum(sid - 10, 0)

    pltpu.sync_copy(z2_hbm.at[pl.ds(0, 624)],
                    shared_acc.at[pl.ds(sid * 624, 624)])

    @pl.when(sid == 0)
    def _zero_tail():
        pltpu.sync_copy(z2_hbm.at[pl.ds(0, 16)],
                        shared_acc.at[pl.ds(9984, 16)])
    plsc.subcore_barrier()

    rows = (rows0, rows1)
    exb = (exb0, exb1)
    sem_g = (sem_g0, sem_g1)
    sem_e = (sem_e0, sem_e1)
    sem_s = (sem_s0, sem_s1)

    def refill(bs, sz):
        w1 = pltpu.async_copy(src_hbm.at[pl.ds(cs + bs, sz)],
                              src_blk.at[pl.ds(0, sz)], sem_i)
        w2 = pltpu.async_copy(dst_hbm.at[pl.ds(cs + bs, sz)],
                              dst_blk.at[pl.ds(0, sz)], sem_i)
        w1.wait()
        w2.wait()

    def issue(bs, t, b):
        @pl.when(cid == 0)
        def _():
            pltpu.async_copy(xlo_hbm.at[src_blk.at[t]], rows[b], sem_g[b])

        @pl.when(cid == 1)
        def _():
            pltpu.async_copy(xhi_hbm.at[src_blk.at[t]], rows[b], sem_g[b])

        pltpu.async_copy(ex_hbm.at[pl.ds((cs + bs + t) * _CH, _CH)],
                         exb[b], sem_e[b])

    def drain_gather(t, b):
        pltpu.make_async_copy(xlo_hbm.at[src_blk.at[t]], rows[b],
                              sem_g[b]).wait()
        pltpu.make_async_copy(ex_hbm.at[pl.ds(0, _CH)], exb[b],
                              sem_e[b]).wait()

    def drain_scatter(t, b):
        pltpu.make_async_copy(rows[b], shared_acc.at[dst_blk.at[t]],
                              sem_s[b]).wait()

    def run_block(bs, nblk):
        issue(bs, 0, 0)

        def outer(t2, _):
            for b in range(2):
                t = 2 * t2 + b

                @pl.when(t < nblk)
                def _():
                    @pl.when(t >= 1)
                    def _():
                        drain_scatter(t - 1, 1 - b)

                    @pl.when(t + 1 < nblk)
                    def _():
                        issue(bs, t + 1, 1 - b)

                    drain_gather(t, b)

                    def group(g, _):
                        gb = g * 16
                        exg = exb[b][pl.ds(gb, 16)]
                        for j in range(16):
                            e = gb + j
                            exe = _bcast(exg, j)
                            for q in range(8):
                                rows[b][e, pl.ds(q * 16, 16)] = (
                                    rows[b][e, pl.ds(q * 16, 16)] * exe)
                        return 0

                    lax.fori_loop(0, _CH // 16, group, 0)
                    pltpu.async_copy(rows[b], shared_acc.at[dst_blk.at[t]],
                                     sem_s[b], add=True)
            return 0

        lax.fori_loop(0, 16, outer, 0)
        drain_scatter(nblk - 1, 1)

    for blk in range(3):
        refill(blk * 32, 32)
        run_block(blk * 32, jnp.int32(32))

    @pl.when(sid < 10)
    def _last_full():
        refill(96, 32)
        run_block(96, jnp.int32(32))

    @pl.when(sid >= 10)
    def _last_small():
        refill(96, 24)
        run_block(96, jnp.int32(24))

    plsc.subcore_barrier()

    @pl.when(sid == 0)
    def _out():
        pltpu.sync_copy(shared_acc, out_hbm.at[cid])


def _post_body(agg_ref, den_ref, h_ref, bo_ref, g_ref, b_ref, o_ref):
    den = den_ref[:, 0:1] + den_ref[:, 1:2]
    agg = jnp.concatenate([agg_ref[0], agg_ref[1]], axis=-1)
    conv = agg / (den + 1e-16) + bo_ref[...]
    t = conv + h_ref[...]
    mu = jnp.mean(t, axis=-1, keepdims=True)
    var = jnp.mean((t - mu) ** 2, axis=-1, keepdims=True)
    y = (t - mu) * lax.rsqrt(var + 1e-5) * g_ref[...] + b_ref[...]
    o_ref[...] = jnp.maximum(y, 0.0)


def _post(agg2, den_t, h, bo, g, b):
    BN = 1000
    return pl.pallas_call(
        _post_body,
        grid=(_N // BN,),
        in_specs=[pl.BlockSpec((2, BN, 128), lambda i: (0, i, 0)),
                  pl.BlockSpec((BN, 2), lambda i: (i, 0)),
                  pl.BlockSpec((BN, _D), lambda i: (i, 0)),
                  pl.BlockSpec((1, _D), lambda i: (0, 0)),
                  pl.BlockSpec((1, _D), lambda i: (0, 0)),
                  pl.BlockSpec((1, _D), lambda i: (0, 0))],
        out_specs=pl.BlockSpec((BN, _D), lambda i: (i, 0)),
        out_shape=jax.ShapeDtypeStruct((_N, _D), _F32),
    )(agg2, den_t, h, bo.reshape(1, _D), g.reshape(1, _D), b.reshape(1, _D))


def _postmm_body(agg_ref, den_ref, h_ref, bo_ref, g_ref, b_ref,
                 wl_ref, bl_ref, wr_ref, br_ref,
                 ho_ref, xl_ref, xr_ref, xlo_ref, xhi_ref):
    den = den_ref[:, 0:1] + den_ref[:, 1:2]
    agg = jnp.concatenate([agg_ref[0], agg_ref[1]], axis=-1)
    conv = agg / (den + 1e-16) + bo_ref[...]
    t = conv + h_ref[...]
    mu = jnp.mean(t, axis=-1, keepdims=True)
    var = jnp.mean((t - mu) ** 2, axis=-1, keepdims=True)
    y = (t - mu) * lax.rsqrt(var + 1e-5) * g_ref[...] + b_ref[...]
    hn = jnp.maximum(y, 0.0)
    ho_ref[...] = hn
    yl = jnp.dot(hn, wl_ref[...], preferred_element_type=_F32) + bl_ref[...]
    yr = jnp.dot(hn, wr_ref[...], preferred_element_type=_F32) + br_ref[...]
    xl_ref[...] = yl.astype(jnp.bfloat16)
    xr_ref[...] = yr.astype(jnp.bfloat16)
    xlo_ref[...] = yl[:, :128]
    xhi_ref[...] = yl[:, 128:]


def _postmm(agg2, den_t, h, bo, g, b, Wl, bl, Wr, br):
    BN = 1000
    return pl.pallas_call(
        _postmm_body,
        grid=(_N // BN,),
        in_specs=[pl.BlockSpec((2, BN, 128), lambda i: (0, i, 0)),
                  pl.BlockSpec((BN, 2), lambda i: (i, 0)),
                  pl.BlockSpec((BN, _D), lambda i: (i, 0)),
                  pl.BlockSpec((1, _D), lambda i: (0, 0)),
                  pl.BlockSpec((1, _D), lambda i: (0, 0)),
                  pl.BlockSpec((1, _D), lambda i: (0, 0)),
                  pl.BlockSpec((_D, _D), lambda i: (0, 0)),
                  pl.BlockSpec((1, _D), lambda i: (0, 0)),
                  pl.BlockSpec((_D, _D), lambda i: (0, 0)),
                  pl.BlockSpec((1, _D), lambda i: (0, 0))],
        out_specs=[pl.BlockSpec((BN, _D), lambda i: (i, 0)),
                   pl.BlockSpec((BN, _D), lambda i: (i, 0)),
                   pl.BlockSpec((BN, _D), lambda i: (i, 0)),
                   pl.BlockSpec((BN, 128), lambda i: (i, 0)),
                   pl.BlockSpec((BN, 128), lambda i: (i, 0))],
        out_shape=[jax.ShapeDtypeStruct((_N, _D), _F32),
                   jax.ShapeDtypeStruct((_N, _D), jnp.bfloat16),
                   jax.ShapeDtypeStruct((_N, _D), jnp.bfloat16),
                   jax.ShapeDtypeStruct((_N, 128), _F32),
                   jax.ShapeDtypeStruct((_N, 128), _F32)],
    )(agg2, den_t, h, bo.reshape(1, _D), g.reshape(1, _D), b.reshape(1, _D),
      Wl, bl.reshape(1, _D), Wr, br.reshape(1, _D))


def kernel(x, edge_index, edge_attr, params):
    src2 = edge_index[0].reshape(_NCH, _CH)
    dst2 = edge_index[1].reshape(_NCH, _CH)
    ea02 = edge_attr[:, 0].reshape(_NCH, _CH)
    ea12 = edge_attr[:, 1].reshape(_NCH, _CH)
    zf = jnp.zeros((_N,), _F32)
    z2 = jnp.zeros((624, 128), _F32)
    h = x
    xl, xr, xlo, xhi = _mm(h, params["Wl0"], params["bl0"],
                           params["Wr0"], params["br0"])
    for k in range(3):
        att = params[f"att{k}"]
        wa = jnp.concatenate([params[f"We{k}"], 0.6 * att, 0.4 * att], axis=0)
        wa = wa.reshape(4, 8, 16, 2).transpose(0, 1, 3, 2).reshape(4, _D)
        xlI = lax.bitcast_convert_type(xl.reshape(_N, 128, 2), jnp.int32)
        xrI = lax.bitcast_convert_type(xr.reshape(_N, 128, 2), jnp.int32)
        ex2, den2 = _pass1(xlI, xrI, src2, dst2, ea02, ea12, wa, zf)
        agg2 = _pass2(xlo, xhi, src2, dst2, ex2.reshape(_E), z2)
        if k < 2:
            h, xl, xr, xlo, xhi = _postmm(
                agg2, den2.T, h, params[f"bo{k}"], params[f"ln_g{k}"],
                params[f"ln_b{k}"], params[f"Wl{k+1}"], params[f"bl{k+1}"],
                params[f"Wr{k+1}"], params[f"br{k+1}"])
        else:
            h = _post(agg2, den2.T, h, params[f"bo{k}"], params[f"ln_g{k}"],
                      params[f"ln_b{k}"])
    return h


# speedup vs baseline: 1.7849x; 1.7849x over previous
import functools

import jax
import jax.numpy as jnp
from jax import lax
from jax.experimental import pallas as pl
from jax.experimental.pallas import tpu as pltpu
from jax.experimental.pallas import tpu_sc as plsc

_N = 10000
_E = 160000
_D = 256
_CH = 80
_NCH = _E // _CH
_F32 = jnp.float32

_mesh = plsc.VectorSubcoreMesh(core_axis_name="c", subcore_axis_name="s")
_scp = pltpu.CompilerParams(needs_layout_passes=False)


_GDN = lax.GatherDimensionNumbers(offset_dims=(), collapsed_slice_dims=(0,),
                                 start_index_map=(0,))


def _bcast(vec, j):
    idx = jnp.full((16, 1), j, jnp.int32)
    return lax.gather(vec, idx, _GDN, (1,),
                      mode=lax.GatherScatterMode.PROMISE_IN_BOUNDS)


def _mm_body(h_ref, wl_ref, bl_ref, wr_ref, br_ref,
             xl_ref, xr_ref, xlo_ref, xhi_ref):
    h = h_ref[...]
    yl = jnp.dot(h, wl_ref[...], preferred_element_type=_F32) + bl_ref[...]
    yr = jnp.dot(h, wr_ref[...], preferred_element_type=_F32) + br_ref[...]
    xl_ref[...] = yl
    xr_ref[...] = yr
    xlo_ref[...] = yl[:, :128]
    xhi_ref[...] = yl[:, 128:]


def _mm(h, Wl, bl, Wr, br):
    BN = 1000
    return pl.pallas_call(
        _mm_body,
        grid=(_N // BN,),
        in_specs=[pl.BlockSpec((BN, _D), lambda i: (i, 0)),
                  pl.BlockSpec((_D, _D), lambda i: (0, 0)),
                  pl.BlockSpec((1, _D), lambda i: (0, 0)),
                  pl.BlockSpec((_D, _D), lambda i: (0, 0)),
                  pl.BlockSpec((1, _D), lambda i: (0, 0))],
        out_specs=[pl.BlockSpec((BN, _D), lambda i: (i, 0)),
                   pl.BlockSpec((BN, _D), lambda i: (i, 0)),
                   pl.BlockSpec((BN, 128), lambda i: (i, 0)),
                   pl.BlockSpec((BN, 128), lambda i: (i, 0))],
        out_shape=[jax.ShapeDtypeStruct((_N, _D), _F32),
                   jax.ShapeDtypeStruct((_N, _D), _F32),
                   jax.ShapeDtypeStruct((_N, 128), _F32),
                   jax.ShapeDtypeStruct((_N, 128), _F32)],
    )(h, Wl, bl.reshape(1, _D), Wr, br.reshape(1, _D))


@functools.partial(
    pl.kernel, mesh=_mesh, compiler_params=_scp,
    out_type=[jax.ShapeDtypeStruct((_NCH, _CH), _F32),
              jax.ShapeDtypeStruct((2, _N), _F32)],
    scratch_types=[
        pltpu.VMEM((64, _CH), jnp.int32),
        pltpu.VMEM((64, _CH), jnp.int32),
        pltpu.VMEM((64, _CH), _F32),
        pltpu.VMEM((64, _CH), _F32),
        pltpu.VMEM((64, _CH), _F32),
        pltpu.VMEM((_CH, _D), _F32),
        pltpu.VMEM((_CH, _D), _F32),
        pltpu.VMEM((_CH, _D), _F32),
        pltpu.VMEM((_CH, _D), _F32),
        pltpu.VMEM((3, _D), _F32),
        pltpu.VMEM_SHARED((_N,), _F32),
        pltpu.SemaphoreType.DMA,
        pltpu.SemaphoreType.DMA,
        pltpu.SemaphoreType.DMA,
        pltpu.SemaphoreType.DMA,
        pltpu.SemaphoreType.DMA,
        pltpu.SemaphoreType.DMA,
        pltpu.SemaphoreType.DMA,
    ])
def _pass1(xl_hbm, xr_hbm, src_hbm, dst_hbm, ea0_hbm, ea1_hbm, wa_hbm,
           zf_hbm, ex_hbm, den_hbm,
           src_blk, dst_blk, ea0_blk, ea1_blk, ex_blk,
           xl0, xl1, xr0, xr1, wa_v,
           shared_den, sem_i, sem_xl0, sem_xl1, sem_xr0, sem_xr1,
           sem_d0, sem_d1):
    cid = lax.axis_index("c")
    sid = lax.axis_index("s")
    wid = sid * 2 + cid
    nt = jnp.where(wid < 26, 64, 56)
    cstart = wid * 64 - 8 * jnp.maximum(wid - 26, 0)

    @pl.when(sid == 0)
    def _zero():
        pltpu.sync_copy(zf_hbm, shared_den)

    pairs = ((src_hbm, src_blk), (dst_hbm, dst_blk),
             (ea0_hbm, ea0_blk), (ea1_hbm, ea1_blk))

    @pl.when(wid < 26)
    def _copy_full():
        waits = [pltpu.async_copy(h.at[pl.ds(cstart, 64)],
                                  v.at[pl.ds(0, 64)], sem_i)
                 for h, v in pairs]
        for w in waits:
            w.wait()

    @pl.when(wid >= 26)
    def _copy_small():
        waits = [pltpu.async_copy(h.at[pl.ds(cstart, 56)],
                                  v.at[pl.ds(0, 56)], sem_i)
                 for h, v in pairs]
        for w in waits:
            w.wait()

    pltpu.sync_copy(wa_hbm, wa_v)

    plsc.subcore_barrier()

    we0 = [wa_v[0, pl.ds(v * 16, 16)] for v in range(16)]
    we1 = [wa_v[1, pl.ds(v * 16, 16)] for v in range(16)]
    attv = [wa_v[2, pl.ds(v * 16, 16)] for v in range(16)]
    lane = lax.iota(jnp.int32, 16)

    bufs = ((xl0, xr0, sem_xl0, sem_xr0), (xl1, xr1, sem_xl1, sem_xr1))
    sem_d = (sem_d0, sem_d1)

    def issue(t, b):
        xlb, xrb, sl, sr = bufs[b]
        pltpu.async_copy(xl_hbm.at[src_blk.at[t]], xlb, sl)
        pltpu.async_copy(xr_hbm.at[dst_blk.at[t]], xrb, sr)

    def drain(t, b):
        xlb, xrb, sl, sr = bufs[b]
        pltpu.make_async_copy(xl_hbm.at[src_blk.at[t]], xlb, sl).wait()
        pltpu.make_async_copy(xr_hbm.at[dst_blk.at[t]], xrb, sr).wait()

    def compute(t, b):
        xlb, xrb, _, _ = bufs[b]

        @plsc.parallel_loop(0, _CH // 16, step=1)
        def group(g):
            gb = g * 16
            ea0g = ea0_blk[t, pl.ds(gb, 16)]
            ea1g = ea1_blk[t, pl.ds(gb, 16)]
            alpha_g = jnp.zeros((16,), _F32)
            for j in range(16):
                e = gb + j
                ea0 = _bcast(ea0g, j)
                ea1 = _bcast(ea1g, j)
                acc_l = jnp.zeros((16,), _F32)
                acc_a = jnp.zeros((16,), _F32)
                for v in range(16):
                    mv = xlb[e, pl.ds(v * 16, 16)] + xrb[e, pl.ds(v * 16, 16)]
                    mv = ea0 * we0[v] + mv
                    mv = ea1 * we1[v] + mv
                    acc_l = acc_l + mv * attv[v]
                    acc_a = acc_a + jnp.abs(mv) * attv[v]
                alpha = plsc.cumsum(0.6 * acc_l + 0.4 * acc_a)[15]
                alpha_g = jnp.where(lane == j, alpha, alpha_g)
            ex_blk[t, pl.ds(gb, 16)] = jnp.exp(alpha_g)

        pltpu.async_copy(ex_blk.at[t], shared_den.at[dst_blk.at[t]],
                         sem_d[b], add=True)

    def drain_denom(t, b):
        pltpu.make_async_copy(ex_blk.at[t], shared_den.at[dst_blk.at[t]],
                              sem_d[b]).wait()

    issue(0, 0)

    def outer(t2, _):
        for b in range(2):
            t = 2 * t2 + b

            @pl.when(t < nt)
            def _():
                @pl.when(t + 1 < nt)
                def _():
                    issue(t + 1, 1 - b)

                drain(t, b)

                @pl.when(t >= 2)
                def _():
                    drain_denom(t - 2, b)

                compute(t, b)
        return 0

    lax.fori_loop(0, 32, outer, 0)
    drain_denom(nt - 2, 0)
    drain_denom(nt - 1, 1)

    @pl.when(wid < 26)
    def _ex_full():
        pltpu.sync_copy(ex_blk.at[pl.ds(0, 64)],
                        ex_hbm.at[pl.ds(cstart, 64)])

    @pl.when(wid >= 26)
    def _ex_small():
        pltpu.sync_copy(ex_blk.at[pl.ds(0, 56)],
                        ex_hbm.at[pl.ds(cstart, 56)])

    plsc.subcore_barrier()

    @pl.when(sid == 0)
    def _out():
        pltpu.sync_copy(shared_den, den_hbm.at[cid])


@functools.partial(
    pl.kernel, mesh=_mesh, compiler_params=_scp,
    out_type=jax.ShapeDtypeStruct((2, _N, 128), _F32),
    scratch_types=[
        pltpu.VMEM((32, _CH), jnp.int32),
        pltpu.VMEM((32, _CH), jnp.int32),
        pltpu.VMEM((_CH,), _F32),
        pltpu.VMEM((_CH,), _F32),
        pltpu.VMEM((_CH, 128), _F32),
        pltpu.VMEM((_CH, 128), _F32),
        pltpu.VMEM_SHARED((_N, 128), _F32),
        pltpu.SemaphoreType.DMA,
        pltpu.SemaphoreType.DMA,
        pltpu.SemaphoreType.DMA,
        pltpu.SemaphoreType.DMA,
        pltpu.SemaphoreType.DMA,
        pltpu.SemaphoreType.DMA,
        pltpu.SemaphoreType.DMA,
    ])
def _pass2(xlo_hbm, xhi_hbm, src_hbm, dst_hbm, ex_hbm, z2_hbm,
           out_hbm,
           src_blk, dst_blk, exb0, exb1, rows0, rows1, shared_acc,
           sem_i, sem_g0, sem_g1, sem_e0, sem_e1, sem_s0, sem_s1):
    cid = lax.axis_index("c")
    sid = lax.axis_index("s")
    cs = sid * 128 - 8 * jnp.maximum(sid - 10, 0)

    pltpu.sync_copy(z2_hbm.at[pl.ds(0, 624)],
                    shared_acc.at[pl.ds(sid * 624, 624)])

    @pl.when(sid == 0)
    def _zero_tail():
        pltpu.sync_copy(z2_hbm.at[pl.ds(0, 16)],
                        shared_acc.at[pl.ds(9984, 16)])
    plsc.subcore_barrier()

    rows = (rows0, rows1)
    exb = (exb0, exb1)
    sem_g = (sem_g0, sem_g1)
    sem_e = (sem_e0, sem_e1)
    sem_s = (sem_s0, sem_s1)

    def refill(bs, sz):
        w1 = pltpu.async_copy(src_hbm.at[pl.ds(cs + bs, sz)],
                              src_blk.at[pl.ds(0, sz)], sem_i)
        w2 = pltpu.async_copy(dst_hbm.at[pl.ds(cs + bs, sz)],
                              dst_blk.at[pl.ds(0, sz)], sem_i)
        w1.wait()
        w2.wait()

    def issue(bs, t, b):
        @pl.when(cid == 0)
        def _():
            pltpu.async_copy(xlo_hbm.at[src_blk.at[t]], rows[b], sem_g[b])

        @pl.when(cid == 1)
        def _():
            pltpu.async_copy(xhi_hbm.at[src_blk.at[t]], rows[b], sem_g[b])

        pltpu.async_copy(ex_hbm.at[pl.ds((cs + bs + t) * _CH, _CH)],
                         exb[b], sem_e[b])

    def drain_gather(t, b):
        pltpu.make_async_copy(xlo_hbm.at[src_blk.at[t]], rows[b],
                              sem_g[b]).wait()
        pltpu.make_async_copy(ex_hbm.at[pl.ds(0, _CH)], exb[b],
                              sem_e[b]).wait()

    def drain_scatter(t, b):
        pltpu.make_async_copy(rows[b], shared_acc.at[dst_blk.at[t]],
                              sem_s[b]).wait()

    def run_block(bs, nblk):
        issue(bs, 0, 0)

        def outer(t2, _):
            for b in range(2):
                t = 2 * t2 + b

                @pl.when(t < nblk)
                def _():
                    @pl.when(t >= 1)
                    def _():
                        drain_scatter(t - 1, 1 - b)

                    @pl.when(t + 1 < nblk)
                    def _():
                        issue(bs, t + 1, 1 - b)

                    drain_gather(t, b)

                    def group(g, _):
                        gb = g * 16
                        exg = exb[b][pl.ds(gb, 16)]
                        for j in range(16):
                            e = gb + j
                            exe = _bcast(exg, j)
                            for q in range(8):
                                rows[b][e, pl.ds(q * 16, 16)] = (
                                    rows[b][e, pl.ds(q * 16, 16)] * exe)
                        return 0

                    lax.fori_loop(0, _CH // 16, group, 0)
                    pltpu.async_copy(rows[b], shared_acc.at[dst_blk.at[t]],
                                     sem_s[b], add=True)
            return 0

        lax.fori_loop(0, 16, outer, 0)
        drain_scatter(nblk - 1, 1)

    for blk in range(3):
        refill(blk * 32, 32)
        run_block(blk * 32, jnp.int32(32))

    @pl.when(sid < 10)
    def _last_full():
        refill(96, 32)
        run_block(96, jnp.int32(32))

    @pl.when(sid >= 10)
    def _last_small():
        refill(96, 24)
        run_block(96, jnp.int32(24))

    plsc.subcore_barrier()

    @pl.when(sid == 0)
    def _out():
        pltpu.sync_copy(shared_acc, out_hbm.at[cid])


def _post_body(agg_ref, den_ref, h_ref, bo_ref, g_ref, b_ref, o_ref):
    den = den_ref[:, 0:1] + den_ref[:, 1:2]
    agg = jnp.concatenate([agg_ref[0], agg_ref[1]], axis=-1)
    conv = agg / (den + 1e-16) + bo_ref[...]
    t = conv + h_ref[...]
    mu = jnp.mean(t, axis=-1, keepdims=True)
    var = jnp.mean((t - mu) ** 2, axis=-1, keepdims=True)
    y = (t - mu) * lax.rsqrt(var + 1e-5) * g_ref[...] + b_ref[...]
    o_ref[...] = jnp.maximum(y, 0.0)


def _post(agg2, den_t, h, bo, g, b):
    BN = 1000
    return pl.pallas_call(
        _post_body,
        grid=(_N // BN,),
        in_specs=[pl.BlockSpec((2, BN, 128), lambda i: (0, i, 0)),
                  pl.BlockSpec((BN, 2), lambda i: (i, 0)),
                  pl.BlockSpec((BN, _D), lambda i: (i, 0)),
                  pl.BlockSpec((1, _D), lambda i: (0, 0)),
                  pl.BlockSpec((1, _D), lambda i: (0, 0)),
                  pl.BlockSpec((1, _D), lambda i: (0, 0))],
        out_specs=pl.BlockSpec((BN, _D), lambda i: (i, 0)),
        out_shape=jax.ShapeDtypeStruct((_N, _D), _F32),
    )(agg2, den_t, h, bo.reshape(1, _D), g.reshape(1, _D), b.reshape(1, _D))


def _postmm_body(agg_ref, den_ref, h_ref, bo_ref, g_ref, b_ref,
                 wl_ref, bl_ref, wr_ref, br_ref,
                 ho_ref, xl_ref, xr_ref, xlo_ref, xhi_ref):
    den = den_ref[:, 0:1] + den_ref[:, 1:2]
    agg = jnp.concatenate([agg_ref[0], agg_ref[1]], axis=-1)
    conv = agg / (den + 1e-16) + bo_ref[...]
    t = conv + h_ref[...]
    mu = jnp.mean(t, axis=-1, keepdims=True)
    var = jnp.mean((t - mu) ** 2, axis=-1, keepdims=True)
    y = (t - mu) * lax.rsqrt(var + 1e-5) * g_ref[...] + b_ref[...]
    hn = jnp.maximum(y, 0.0)
    ho_ref[...] = hn
    yl = jnp.dot(hn, wl_ref[...], preferred_element_type=_F32) + bl_ref[...]
    yr = jnp.dot(hn, wr_ref[...], preferred_element_type=_F32) + br_ref[...]
    xl_ref[...] = yl
    xr_ref[...] = yr
    xlo_ref[...] = yl[:, :128]
    xhi_ref[...] = yl[:, 128:]


def _postmm(agg2, den_t, h, bo, g, b, Wl, bl, Wr, br):
    BN = 1000
    return pl.pallas_call(
        _postmm_body,
        grid=(_N // BN,),
        in_specs=[pl.BlockSpec((2, BN, 128), lambda i: (0, i, 0)),
                  pl.BlockSpec((BN, 2), lambda i: (i, 0)),
                  pl.BlockSpec((BN, _D), lambda i: (i, 0)),
                  pl.BlockSpec((1, _D), lambda i: (0, 0)),
                  pl.BlockSpec((1, _D), lambda i: (0, 0)),
                  pl.BlockSpec((1, _D), lambda i: (0, 0)),
                  pl.BlockSpec((_D, _D), lambda i: (0, 0)),
                  pl.BlockSpec((1, _D), lambda i: (0, 0)),
                  pl.BlockSpec((_D, _D), lambda i: (0, 0)),
                  pl.BlockSpec((1, _D), lambda i: (0, 0))],
        out_specs=[pl.BlockSpec((BN, _D), lambda i: (i, 0)),
                   pl.BlockSpec((BN, _D), lambda i: (i, 0)),
                   pl.BlockSpec((BN, _D), lambda i: (i, 0)),
                   pl.BlockSpec((BN, 128), lambda i: (i, 0)),
                   pl.BlockSpec((BN, 128), lambda i: (i, 0))],
        out_shape=[jax.ShapeDtypeStruct((_N, _D), _F32),
                   jax.ShapeDtypeStruct((_N, _D), _F32),
                   jax.ShapeDtypeStruct((_N, _D), _F32),
                   jax.ShapeDtypeStruct((_N, 128), _F32),
                   jax.ShapeDtypeStruct((_N, 128), _F32)],
    )(agg2, den_t, h, bo.reshape(1, _D), g.reshape(1, _D), b.reshape(1, _D),
      Wl, bl.reshape(1, _D), Wr, br.reshape(1, _D))


def kernel(x, edge_index, edge_attr, params):
    src2 = edge_index[0].reshape(_NCH, _CH)
    dst2 = edge_index[1].reshape(_NCH, _CH)
    ea02 = edge_attr[:, 0].reshape(_NCH, _CH)
    ea12 = edge_attr[:, 1].reshape(_NCH, _CH)
    zf = jnp.zeros((_N,), _F32)
    z2 = jnp.zeros((624, 128), _F32)
    h = x
    xl, xr, xlo, xhi = _mm(h, params["Wl0"], params["bl0"],
                           params["Wr0"], params["br0"])
    for k in range(3):
        wa = jnp.concatenate([params[f"We{k}"], params[f"att{k}"]], axis=0)
        ex2, den2 = _pass1(xl, xr, src2, dst2, ea02, ea12, wa, zf)
        agg2 = _pass2(xlo, xhi, src2, dst2, ex2.reshape(_E), z2)
        if k < 2:
            h, xl, xr, xlo, xhi = _postmm(
                agg2, den2.T, h, params[f"bo{k}"], params[f"ln_g{k}"],
                params[f"ln_b{k}"], params[f"Wl{k+1}"], params[f"bl{k+1}"],
                params[f"Wr{k+1}"], params[f"br{k+1}"])
        else:
            h = _post(agg2, den2.T, h, params[f"bo{k}"], params[f"ln_g{k}"],
                      params[f"ln_b{k}"])
    return h


# speedup vs baseline: 1.7910x; 1.0034x over previous
import functools

import jax
import jax.numpy as jnp
from jax import lax
from jax.experimental import pallas as pl
from jax.experimental.pallas import tpu as pltpu
from jax.experimental.pallas import tpu_sc as plsc

_N = 10000
_E = 160000
_D = 256
_CH = 80
_NCH = _E // _CH
_F32 = jnp.float32

_mesh = plsc.VectorSubcoreMesh(core_axis_name="c", subcore_axis_name="s")
_scp = pltpu.CompilerParams(needs_layout_passes=False)


_GDN = lax.GatherDimensionNumbers(offset_dims=(), collapsed_slice_dims=(0,),
                                 start_index_map=(0,))


def _bcast(vec, j):
    idx = jnp.full((16, 1), j, jnp.int32)
    return lax.gather(vec, idx, _GDN, (1,),
                      mode=lax.GatherScatterMode.PROMISE_IN_BOUNDS)


def _mm_body(h_ref, wl_ref, bl_ref, wr_ref, br_ref,
             xl_ref, xr_ref, xlo_ref, xhi_ref):
    h = h_ref[...]
    yl = jnp.dot(h, wl_ref[...], preferred_element_type=_F32) + bl_ref[...]
    yr = jnp.dot(h, wr_ref[...], preferred_element_type=_F32) + br_ref[...]
    xl_ref[...] = yl
    xr_ref[...] = yr
    xlo_ref[...] = yl[:, :128]
    xhi_ref[...] = yl[:, 128:]


def _mm(h, Wl, bl, Wr, br):
    BN = 1000
    return pl.pallas_call(
        _mm_body,
        grid=(_N // BN,),
        in_specs=[pl.BlockSpec((BN, _D), lambda i: (i, 0)),
                  pl.BlockSpec((_D, _D), lambda i: (0, 0)),
                  pl.BlockSpec((1, _D), lambda i: (0, 0)),
                  pl.BlockSpec((_D, _D), lambda i: (0, 0)),
                  pl.BlockSpec((1, _D), lambda i: (0, 0))],
        out_specs=[pl.BlockSpec((BN, _D), lambda i: (i, 0)),
                   pl.BlockSpec((BN, _D), lambda i: (i, 0)),
                   pl.BlockSpec((BN, 128), lambda i: (i, 0)),
                   pl.BlockSpec((BN, 128), lambda i: (i, 0))],
        out_shape=[jax.ShapeDtypeStruct((_N, _D), _F32),
                   jax.ShapeDtypeStruct((_N, _D), _F32),
                   jax.ShapeDtypeStruct((_N, 128), _F32),
                   jax.ShapeDtypeStruct((_N, 128), _F32)],
    )(h, Wl, bl.reshape(1, _D), Wr, br.reshape(1, _D))


@functools.partial(
    pl.kernel, mesh=_mesh, compiler_params=_scp,
    out_type=[jax.ShapeDtypeStruct((_NCH, _CH), _F32),
              jax.ShapeDtypeStruct((2, _N), _F32)],
    scratch_types=[
        pltpu.VMEM((64, _CH), jnp.int32),
        pltpu.VMEM((64, _CH), jnp.int32),
        pltpu.VMEM((64, _CH), _F32),
        pltpu.VMEM((64, _CH), _F32),
        pltpu.VMEM((64, _CH), _F32),
        pltpu.VMEM((_CH, _D), _F32),
        pltpu.VMEM((_CH, _D), _F32),
        pltpu.VMEM((_CH, _D), _F32),
        pltpu.VMEM((_CH, _D), _F32),
        pltpu.VMEM((3, _D), _F32),
        pltpu.VMEM_SHARED((_N,), _F32),
        pltpu.SemaphoreType.DMA,
        pltpu.SemaphoreType.DMA,
        pltpu.SemaphoreType.DMA,
        pltpu.SemaphoreType.DMA,
        pltpu.SemaphoreType.DMA,
        pltpu.SemaphoreType.DMA,
        pltpu.SemaphoreType.DMA,
    ])
def _pass1(xl_hbm, xr_hbm, src_hbm, dst_hbm, ea0_hbm, ea1_hbm, wa_hbm,
           zf_hbm, ex_hbm, den_hbm,
           src_blk, dst_blk, ea0_blk, ea1_blk, ex_blk,
           xl0, xl1, xr0, xr1, wa_v,
           shared_den, sem_i, sem_xl0, sem_xl1, sem_xr0, sem_xr1,
           sem_d0, sem_d1):
    cid = lax.axis_index("c")
    sid = lax.axis_index("s")
    wid = sid * 2 + cid
    nt = jnp.where(wid < 26, 64, 56)
    cstart = wid * 64 - 8 * jnp.maximum(wid - 26, 0)

    @pl.when(sid == 0)
    def _zero():
        pltpu.sync_copy(zf_hbm, shared_den)

    pairs = ((src_hbm, src_blk), (dst_hbm, dst_blk),
             (ea0_hbm, ea0_blk), (ea1_hbm, ea1_blk))

    @pl.when(wid < 26)
    def _copy_full():
        waits = [pltpu.async_copy(h.at[pl.ds(cstart, 64)],
                                  v.at[pl.ds(0, 64)], sem_i)
                 for h, v in pairs]
        for w in waits:
            w.wait()

    @pl.when(wid >= 26)
    def _copy_small():
        waits = [pltpu.async_copy(h.at[pl.ds(cstart, 56)],
                                  v.at[pl.ds(0, 56)], sem_i)
                 for h, v in pairs]
        for w in waits:
            w.wait()

    pltpu.sync_copy(wa_hbm, wa_v)

    plsc.subcore_barrier()

    we0 = [wa_v[0, pl.ds(v * 16, 16)] for v in range(16)]
    we1 = [wa_v[1, pl.ds(v * 16, 16)] for v in range(16)]
    attv = [wa_v[2, pl.ds(v * 16, 16)] for v in range(16)]
    lane = lax.iota(jnp.int32, 16)

    bufs = ((xl0, xr0, sem_xl0, sem_xr0), (xl1, xr1, sem_xl1, sem_xr1))
    sem_d = (sem_d0, sem_d1)

    def issue(t, b):
        xlb, xrb, sl, sr = bufs[b]
        pltpu.async_copy(xl_hbm.at[src_blk.at[t]], xlb, sl)
        pltpu.async_copy(xr_hbm.at[dst_blk.at[t]], xrb, sr)

    def drain(t, b):
        xlb, xrb, sl, sr = bufs[b]
        pltpu.make_async_copy(xl_hbm.at[src_blk.at[t]], xlb, sl).wait()
        pltpu.make_async_copy(xr_hbm.at[dst_blk.at[t]], xrb, sr).wait()

    def compute(t, b):
        xlb, xrb, _, _ = bufs[b]

        def group(g, _):
            gb = g * 16
            ea0g = ea0_blk[t, pl.ds(gb, 16)]
            ea1g = ea1_blk[t, pl.ds(gb, 16)]
            alpha_g = jnp.zeros((16,), _F32)
            for j in range(16):
                e = gb + j
                ea0 = _bcast(ea0g, j)
                ea1 = _bcast(ea1g, j)
                acc_l = jnp.zeros((16,), _F32)
                acc_a = jnp.zeros((16,), _F32)
                for v in range(16):
                    mv = xlb[e, pl.ds(v * 16, 16)] + xrb[e, pl.ds(v * 16, 16)]
                    mv = ea0 * we0[v] + mv
                    mv = ea1 * we1[v] + mv
                    acc_l = acc_l + mv * attv[v]
                    acc_a = acc_a + jnp.abs(mv) * attv[v]
                alpha = plsc.cumsum(0.6 * acc_l + 0.4 * acc_a)[15]
                alpha_g = jnp.where(lane == j, alpha, alpha_g)
            ex_blk[t, pl.ds(gb, 16)] = jnp.exp(alpha_g)
            return 0

        lax.fori_loop(0, _CH // 16, group, 0)
        pltpu.async_copy(ex_blk.at[t], shared_den.at[dst_blk.at[t]],
                         sem_d[b], add=True)

    def drain_denom(t, b):
        pltpu.make_async_copy(ex_blk.at[t], shared_den.at[dst_blk.at[t]],
                              sem_d[b]).wait()

    issue(0, 0)

    def outer(t2, _):
        for b in range(2):
            t = 2 * t2 + b

            @pl.when(t < nt)
            def _():
                @pl.when(t + 1 < nt)
                def _():
                    issue(t + 1, 1 - b)

                drain(t, b)

                @pl.when(t >= 2)
                def _():
                    drain_denom(t - 2, b)

                compute(t, b)
        return 0

    lax.fori_loop(0, 32, outer, 0)
    drain_denom(nt - 2, 0)
    drain_denom(nt - 1, 1)

    @pl.when(wid < 26)
    def _ex_full():
        pltpu.sync_copy(ex_blk.at[pl.ds(0, 64)],
                        ex_hbm.at[pl.ds(cstart, 64)])

    @pl.when(wid >= 26)
    def _ex_small():
        pltpu.sync_copy(ex_blk.at[pl.ds(0, 56)],
                        ex_hbm.at[pl.ds(cstart, 56)])

    plsc.subcore_barrier()

    @pl.when(sid == 0)
    def _out():
        pltpu.sync_copy(shared_den, den_hbm.at[cid])


@functools.partial(
    pl.kernel, mesh=_mesh, compiler_params=_scp,
    out_type=jax.ShapeDtypeStruct((2, _N, 128), _F32),
    scratch_types=[
        pltpu.VMEM((32, _CH), jnp.int32),
        pltpu.VMEM((32, _CH), jnp.int32),
        pltpu.VMEM((_CH,), _F32),
        pltpu.VMEM((_CH,), _F32),
        pltpu.VMEM((_CH, 128), _F32),
        pltpu.VMEM((_CH, 128), _F32),
        pltpu.VMEM_SHARED((_N, 128), _F32),
        pltpu.SemaphoreType.DMA,
        pltpu.SemaphoreType.DMA,
        pltpu.SemaphoreType.DMA,
        pltpu.SemaphoreType.DMA,
        pltpu.SemaphoreType.DMA,
        pltpu.SemaphoreType.DMA,
        pltpu.SemaphoreType.DMA,
    ])
def _pass2(xlo_hbm, xhi_hbm, src_hbm, dst_hbm, ex_hbm, z2_hbm,
           out_hbm,
           src_blk, dst_blk, exb0, exb1, rows0, rows1, shared_acc,
           sem_i, sem_g0, sem_g1, sem_e0, sem_e1, sem_s0, sem_s1):
    cid = lax.axis_index("c")
    sid = lax.axis_index("s")
    cs = sid * 128 - 8 * jnp.maximum(sid - 10, 0)

    pltpu.sync_copy(z2_hbm.at[pl.ds(0, 624)],
                    shared_acc.at[pl.ds(sid * 624, 624)])

    @pl.when(sid == 0)
    def _zero_tail():
        pltpu.sync_copy(z2_hbm.at[pl.ds(0, 16)],
                        shared_acc.at[pl.ds(9984, 16)])
    plsc.subcore_barrier()

    rows = (rows0, rows1)
    exb = (exb0, exb1)
    sem_g = (sem_g0, sem_g1)
    sem_e = (sem_e0, sem_e1)
    sem_s = (sem_s0, sem_s1)

    def refill(bs, sz):
        w1 = pltpu.async_copy(src_hbm.at[pl.ds(cs + bs, sz)],
                              src_blk.at[pl.ds(0, sz)], sem_i)
        w2 = pltpu.async_copy(dst_hbm.at[pl.ds(cs + bs, sz)],
                              dst_blk.at[pl.ds(0, sz)], sem_i)
        w1.wait()
        w2.wait()

    def issue(bs, t, b):
        @pl.when(cid == 0)
        def _():
            pltpu.async_copy(xlo_hbm.at[src_blk.at[t]], rows[b], sem_g[b])

        @pl.when(cid == 1)
        def _():
            pltpu.async_copy(xhi_hbm.at[src_blk.at[t]], rows[b], sem_g[b])

        pltpu.async_copy(ex_hbm.at[pl.ds((cs + bs + t) * _CH, _CH)],
                         exb[b], sem_e[b])

    def drain_gather(t, b):
        pltpu.make_async_copy(xlo_hbm.at[src_blk.at[t]], rows[b],
                              sem_g[b]).wait()
        pltpu.make_async_copy(ex_hbm.at[pl.ds(0, _CH)], exb[b],
                              sem_e[b]).wait()

    def drain_scatter(t, b):
        pltpu.make_async_copy(rows[b], shared_acc.at[dst_blk.at[t]],
                              sem_s[b]).wait()

    def run_block(bs, nblk):
        issue(bs, 0, 0)

        def outer(t2, _):
            for b in range(2):
                t = 2 * t2 + b

                @pl.when(t < nblk)
                def _():
                    @pl.when(t >= 1)
                    def _():
                        drain_scatter(t - 1, 1 - b)

                    @pl.when(t + 1 < nblk)
                    def _():
                        issue(bs, t + 1, 1 - b)

                    drain_gather(t, b)

                    def group(g, _):
                        gb = g * 16
                        exg = exb[b][pl.ds(gb, 16)]
                        for j in range(16):
                            e = gb + j
                            exe = _bcast(exg, j)
                            for q in range(8):
                                rows[b][e, pl.ds(q * 16, 16)] = (
                                    rows[b][e, pl.ds(q * 16, 16)] * exe)
                        return 0

                    lax.fori_loop(0, _CH // 16, group, 0)
                    pltpu.async_copy(rows[b], shared_acc.at[dst_blk.at[t]],
                                     sem_s[b], add=True)
            return 0

        lax.fori_loop(0, 16, outer, 0)
        drain_scatter(nblk - 1, 1)

    for blk in range(3):
        refill(blk * 32, 32)
        run_block(blk * 32, jnp.int32(32))

    @pl.when(sid < 10)
    def _last_full():
        refill(96, 32)
        run_block(96, jnp.int32(32))

    @pl.when(sid >= 10)
    def _last_small():
        refill(96, 24)
        run_block(96, jnp.int32(24))

    plsc.subcore_barrier()

    @pl.when(sid == 0)
    def _out():
        pltpu.sync_copy(shared_acc, out_hbm.at[cid])


def _post_body(agg_ref, den_ref, h_ref, bo_ref, g_ref, b_ref, o_ref):
    den = den_ref[:, 0:1] + den_ref[:, 1:2]
    agg = jnp.concatenate([agg_ref[0], agg_ref[1]], axis=-1)
    conv = agg / (den + 1e-16) + bo_ref[...]
    t = conv + h_ref[...]
    mu = jnp.mean(t, axis=-1, keepdims=True)
    var = jnp.mean((t - mu) ** 2, axis=-1, keepdims=True)
    y = (t - mu) * lax.rsqrt(var + 1e-5) * g_ref[...] + b_ref[...]
    o_ref[...] = jnp.maximum(y, 0.0)


def _post(agg2, den_t, h, bo, g, b):
    BN = 1000
    return pl.pallas_call(
        _post_body,
        grid=(_N // BN,),
        in_specs=[pl.BlockSpec((2, BN, 128), lambda i: (0, i, 0)),
                  pl.BlockSpec((BN, 2), lambda i: (i, 0)),
                  pl.BlockSpec((BN, _D), lambda i: (i, 0)),
                  pl.BlockSpec((1, _D), lambda i: (0, 0)),
                  pl.BlockSpec((1, _D), lambda i: (0, 0)),
                  pl.BlockSpec((1, _D), lambda i: (0, 0))],
        out_specs=pl.BlockSpec((BN, _D), lambda i: (i, 0)),
        out_shape=jax.ShapeDtypeStruct((_N, _D), _F32),
    )(agg2, den_t, h, bo.reshape(1, _D), g.reshape(1, _D), b.reshape(1, _D))


def _postmm_body(agg_ref, den_ref, h_ref, bo_ref, g_ref, b_ref,
                 wl_ref, bl_ref, wr_ref, br_ref,
                 ho_ref, xl_ref, xr_ref, xlo_ref, xhi_ref):
    den = den_ref[:, 0:1] + den_ref[:, 1:2]
    agg = jnp.concatenate([agg_ref[0], agg_ref[1]], axis=-1)
    conv = agg / (den + 1e-16) + bo_ref[...]
    t = conv + h_ref[...]
    mu = jnp.mean(t, axis=-1, keepdims=True)
    var = jnp.mean((t - mu) ** 2, axis=-1, keepdims=True)
    y = (t - mu) * lax.rsqrt(var + 1e-5) * g_ref[...] + b_ref[...]
    hn = jnp.maximum(y, 0.0)
    ho_ref[...] = hn
    yl = jnp.dot(hn, wl_ref[...], preferred_element_type=_F32) + bl_ref[...]
    yr = jnp.dot(hn, wr_ref[...], preferred_element_type=_F32) + br_ref[...]
    xl_ref[...] = yl
    xr_ref[...] = yr
    xlo_ref[...] = yl[:, :128]
    xhi_ref[...] = yl[:, 128:]


def _postmm(agg2, den_t, h, bo, g, b, Wl, bl, Wr, br):
    BN = 1000
    return pl.pallas_call(
        _postmm_body,
        grid=(_N // BN,),
        in_specs=[pl.BlockSpec((2, BN, 128), lambda i: (0, i, 0)),
                  pl.BlockSpec((BN, 2), lambda i: (i, 0)),
                  pl.BlockSpec((BN, _D), lambda i: (i, 0)),
                  pl.BlockSpec((1, _D), lambda i: (0, 0)),
                  pl.BlockSpec((1, _D), lambda i: (0, 0)),
                  pl.BlockSpec((1, _D), lambda i: (0, 0)),
                  pl.BlockSpec((_D, _D), lambda i: (0, 0)),
                  pl.BlockSpec((1, _D), lambda i: (0, 0)),
                  pl.BlockSpec((_D, _D), lambda i: (0, 0)),
                  pl.BlockSpec((1, _D), lambda i: (0, 0))],
        out_specs=[pl.BlockSpec((BN, _D), lambda i: (i, 0)),
                   pl.BlockSpec((BN, _D), lambda i: (i, 0)),
                   pl.BlockSpec((BN, _D), lambda i: (i, 0)),
                   pl.BlockSpec((BN, 128), lambda i: (i, 0)),
                   pl.BlockSpec((BN, 128), lambda i: (i, 0))],
        out_shape=[jax.ShapeDtypeStruct((_N, _D), _F32),
                   jax.ShapeDtypeStruct((_N, _D), _F32),
                   jax.ShapeDtypeStruct((_N, _D), _F32),
                   jax.ShapeDtypeStruct((_N, 128), _F32),
                   jax.ShapeDtypeStruct((_N, 128), _F32)],
    )(agg2, den_t, h, bo.reshape(1, _D), g.reshape(1, _D), b.reshape(1, _D),
      Wl, bl.reshape(1, _D), Wr, br.reshape(1, _D))


def kernel(x, edge_index, edge_attr, params):
    src2 = edge_index[0].reshape(_NCH, _CH)
    dst2 = edge_index[1].reshape(_NCH, _CH)
    ea02 = edge_attr[:, 0].reshape(_NCH, _CH)
    ea12 = edge_attr[:, 1].reshape(_NCH, _CH)
    zf = jnp.zeros((_N,), _F32)
    z2 = jnp.zeros((624, 128), _F32)
    h = x
    xl, xr, xlo, xhi = _mm(h, params["Wl0"], params["bl0"],
                           params["Wr0"], params["br0"])
    for k in range(3):
        wa = jnp.concatenate([params[f"We{k}"], params[f"att{k}"]], axis=0)
        ex2, den2 = _pass1(xl, xr, src2, dst2, ea02, ea12, wa, zf)
        agg2 = _pass2(xlo, xhi, src2, dst2, ex2.reshape(_E), z2)
        if k < 2:
            h, xl, xr, xlo, xhi = _postmm(
                agg2, den2.T, h, params[f"bo{k}"], params[f"ln_g{k}"],
                params[f"ln_b{k}"], params[f"Wl{k+1}"], params[f"bl{k+1}"],
                params[f"Wr{k+1}"], params[f"br{k+1}"])
        else:
            h = _post(agg2, den2.T, h, params[f"bo{k}"], params[f"ln_g{k}"],
                      params[f"ln_b{k}"])
    return h
